# baseline (speedup 1.0000x reference)
import functools

import jax
import jax.numpy as jnp
from jax import lax
from jax.experimental import pallas as pl
from jax.experimental.pallas import tpu as pltpu
from jax.experimental.pallas import tpu_sc as plsc

NC = 2
NS = 16
NW = NC * NS
L = 16

N = 10000
E = 320000
D = 128
R = 8

CNTP = 81920
CSLICE = CNTP // NS
KC = 80
NBUF = 2
EPT = E // NW
ECT = E // NS
MB = 2000
MC = MB // KC
CHALF = CSLICE // 2
NP = 10240
ROWS_PER_TILE = NP // NS
ZROWS = 16


def _sc_graph_kernel(src, dst, etype, zflat):
  mesh = plsc.VectorSubcoreMesh(core_axis_name="c", subcore_axis_name="s")

  @functools.partial(
      pl.kernel,
      out_type=jax.ShapeDtypeStruct((NC * NP, D), jnp.float32),
      mesh=mesh,
      scratch_types=[
          pltpu.VMEM_SHARED((CNTP,), jnp.float32),
          pltpu.VMEM_SHARED((NP, D), jnp.float32),
          pltpu.VMEM((MB,), jnp.int32),
          pltpu.VMEM((MB,), jnp.int32),
          pltpu.VMEM((MB,), jnp.int32),
          pltpu.VMEM((MC, KC), jnp.int32),
          pltpu.VMEM((MC, KC), jnp.int32),
          pltpu.VMEM((MC, KC), jnp.int32),
          pltpu.VMEM((KC,), jnp.float32),
          [pltpu.VMEM((KC,), jnp.float32)] * NBUF,
          [pltpu.VMEM((KC, D), jnp.float32)] * NBUF,
          pltpu.VMEM((CHALF,), jnp.float32),
          pltpu.VMEM((ZROWS, D), jnp.float32),
          [pltpu.SemaphoreType.DMA] * NBUF,
          [pltpu.SemaphoreType.DMA] * NBUF,
          pltpu.SemaphoreType.DMA,
          pltpu.SemaphoreType.DMA,
      ],
  )
  def k(src_hbm, dst_hbm, typ_hbm, z_hbm, parts_hbm,
        cnt_sp, acc_sp, srcb, dstb, typb, db, gb, hb,
        onesb, scalebs, rowsbs, sliceb, zerob,
        gsems, ssems, wsem, esem):
    cid = lax.axis_index("c")
    sid = lax.axis_index("s")
    zeros = jnp.zeros((L,), jnp.float32)

    @pl.loop(0, ZROWS)
    def _(i):
      for q in range(D // L):
        zerob[i, pl.ds(q * L, L)] = zeros

    @pl.loop(0, CHALF // L)
    def _(i):
      sliceb[pl.ds(i * L, L)] = zeros

    @pl.loop(0, KC // L)
    def _(i):
      onesb[pl.ds(i * L, L)] = jnp.ones((L,), jnp.float32)

    for p in range(2):
      pltpu.sync_copy(sliceb, cnt_sp.at[pl.ds(sid * CSLICE + p * CHALF, CHALF)])

    @pl.loop(0, ROWS_PER_TILE // ZROWS)
    def _(i):
      pltpu.sync_copy(
          zerob, acc_sp.at[pl.ds(sid * ROWS_PER_TILE + i * ZROWS, ZROWS)])

    plsc.subcore_barrier()

    base1 = sid * ECT

    @pl.loop(0, ECT // MB)
    def _(m):
      off = base1 + m * MB
      f1 = pltpu.async_copy(dst_hbm.at[pl.ds(off, MB)], dstb, esem)
      f2 = pltpu.async_copy(typ_hbm.at[pl.ds(off, MB)], typb, esem)
      f1.wait()
      f2.wait()

      @pl.loop(0, MC)
      def _(c):
        for j in range(KC // L):
          sl = pl.ds(c * KC + j * L, L)
          db[c, pl.ds(j * L, L)] = typb[sl] * N + dstb[sl]

      @pl.loop(0, MC // 5)
      def _(w):
        descs = [
            pltpu.async_copy(onesb, cnt_sp.at[db.at[w * 5 + b]], wsem,
                             add=True)
            for b in range(5)
        ]
        for d_ in descs:
          d_.wait()

    plsc.subcore_barrier()

    cbase = sid * CSLICE
    for p in range(2):
      pltpu.sync_copy(cnt_sp.at[pl.ds(cbase + p * CHALF, CHALF)], sliceb)

      @pl.loop(0, CHALF // L)
      def _(i):
        sl = pl.ds(i * L, L)
        sliceb[sl] = 1.0 / jnp.maximum(sliceb[sl], 1.0)

      pltpu.sync_copy(sliceb, cnt_sp.at[pl.ds(cbase + p * CHALF, CHALF)])
    plsc.subcore_barrier()

    base2 = (cid * NS + sid) * EPT

    def chunk_in(c, b):
      sd = pltpu.async_copy(cnt_sp.at[hb.at[c]], scalebs[b], ssems[b])
      rd = pltpu.async_copy(z_hbm.at[gb.at[c]], rowsbs[b], gsems[b])
      return sd, rd

    def chunk_out(c, b, sd, rd):
      sd.wait()
      rd.wait()
      rowsb = rowsbs[b]
      scaleb = scalebs[b]

      @pl.loop(0, KC // L)
      def _(gg):
        sv = scaleb[pl.ds(gg * L, L)]
        for j in range(L):
          s = sv[j]
          row = gg * L + j
          for q in range(D // L):
            sl = pl.ds(q * L, L)
            rowsb[row, sl] = rowsb[row, sl] * s

      return pltpu.async_copy(rowsb, acc_sp.at[db.at[c]], wsem, add=True)

    @pl.loop(0, EPT // MB)
    def _(m):
      off = base2 + m * MB
      f1 = pltpu.async_copy(src_hbm.at[pl.ds(off, MB)], srcb, esem)
      f2 = pltpu.async_copy(dst_hbm.at[pl.ds(off, MB)], dstb, esem)
      f3 = pltpu.async_copy(typ_hbm.at[pl.ds(off, MB)], typb, esem)
      f1.wait()
      f2.wait()
      f3.wait()

      @pl.loop(0, MC)
      def _(c):
        for j in range(KC // L):
          sl = pl.ds(c * KC + j * L, L)
          jj = pl.ds(j * L, L)
          t = typb[sl]
          d_ = dstb[sl]
          gb[c, jj] = (t + 1) * N + srcb[sl]
          hb[c, jj] = t * N + d_
          db[c, jj] = d_

      @pl.loop(0, (MC - 1) // NBUF)
      def _(w):
        c0 = w * NBUF
        ins = [chunk_in(c0 + b, b) for b in range(NBUF)]
        outs = [chunk_out(c0 + b, b, *ins[b]) for b in range(NBUF)]
        for d_ in outs:
          d_.wait()

      sd, rd = chunk_in(MC - 1, 0)
      chunk_out(MC - 1, 0, sd, rd).wait()

    plsc.subcore_barrier()

    rbase = sid * ROWS_PER_TILE
    pltpu.sync_copy(acc_sp.at[pl.ds(rbase, ROWS_PER_TILE)],
                    parts_hbm.at[pl.ds(cid * NP + rbase, ROWS_PER_TILE)])

  return k(src, dst, etype, zflat)


BN = 2000


def _mm_body(x_ref, w_ref, b_ref, z_ref):
  i = pl.program_id(0)
  acc = jnp.dot(x_ref[...], w_ref[0], preferred_element_type=jnp.float32)
  sel = jnp.where(i == 0, 1.0, 0.0).astype(jnp.float32)
  z_ref[0] = acc + sel * b_ref[...]


def _fin_body(z0_ref, p_ref, a_ref, o_ref):
  o = z0_ref[...] + p_ref[0] + p_ref[1]
  o_ref[...] = jnp.where(o > 0, o, a_ref[...] * o)


def kernel(x, edge_index, edge_type, weight, root, bias, prelu_a):
  src = edge_index[0]
  dst = edge_index[1]
  wcat = jnp.concatenate([root[None], weight], axis=0)

  zfull = pl.pallas_call(
      _mm_body,
      grid=(R + 1, N // BN),
      in_specs=[
          pl.BlockSpec((BN, D), lambda i, nb: (nb, 0)),
          pl.BlockSpec((1, D, D), lambda i, nb: (i, 0, 0)),
          pl.BlockSpec((1, D), lambda i, nb: (0, 0)),
      ],
      out_specs=pl.BlockSpec((1, BN, D), lambda i, nb: (i, nb, 0)),
      out_shape=jax.ShapeDtypeStruct((R + 1, N, D), jnp.float32),
  )(x, wcat, bias[None])

  zflat = zfull.reshape((R + 1) * N, D)
  parts = _sc_graph_kernel(src, dst, edge_type, zflat)
  parts = parts.reshape(NC, NP, D)[:, :N]

  return pl.pallas_call(
      _fin_body,
      grid=(N // BN,),
      in_specs=[
          pl.BlockSpec((BN, D), lambda nb: (nb, 0)),
          pl.BlockSpec((NC, BN, D), lambda nb: (0, nb, 0)),
          pl.BlockSpec((1, D), lambda nb: (0, 0)),
      ],
      out_specs=pl.BlockSpec((BN, D), lambda nb: (nb, 0)),
      out_shape=jax.ShapeDtypeStruct((N, D), jnp.float32),
  )(zfull[0], parts, prelu_a[None])


# speedup vs baseline: 1.0925x; 1.0925x over previous
import functools

import jax
import jax.numpy as jnp
from jax import lax
from jax.experimental import pallas as pl
from jax.experimental.pallas import tpu as pltpu
from jax.experimental.pallas import tpu_sc as plsc

NC = 2
NS = 16
NW = NC * NS
L = 16

N = 10000
E = 320000
D = 128
R = 8

CNTP = 81920
CSLICE = CNTP // NS
KC = 80
NBUF = 3
EPT = E // NW
ECT = E // NS
MB = 2000
MC = MB // KC
CHALF = CSLICE // 2
NP = 10240
ROWS_PER_TILE = NP // NS
ZROWS = 16


def _sc_graph_kernel(edge_index, etype, zflat):
  mesh = plsc.VectorSubcoreMesh(core_axis_name="c", subcore_axis_name="s")

  @functools.partial(
      pl.kernel,
      out_type=jax.ShapeDtypeStruct((NC * NP, D), jnp.float32),
      mesh=mesh,
      scratch_types=[
          pltpu.VMEM_SHARED((CNTP,), jnp.float32),
          pltpu.VMEM_SHARED((NP, D), jnp.float32),
          pltpu.VMEM((MB,), jnp.int32),
          pltpu.VMEM((MB,), jnp.int32),
          pltpu.VMEM((MB,), jnp.int32),
          pltpu.VMEM((MC, KC), jnp.int32),
          pltpu.VMEM((KC,), jnp.float32),
          [pltpu.VMEM((KC,), jnp.float32)] * NBUF,
          [pltpu.VMEM((KC, D), jnp.float32)] * NBUF,
          pltpu.VMEM((CHALF,), jnp.float32),
          [pltpu.SemaphoreType.DMA] * NBUF,
          [pltpu.SemaphoreType.DMA] * NBUF,
          pltpu.SemaphoreType.DMA,
          pltpu.SemaphoreType.DMA,
      ],
  )
  def k(ei_hbm, typ_hbm, z_hbm, z1_hbm, z2_hbm, parts_hbm,
        cnt_sp, acc_sp, srcb, dstb, typb, db,
        onesb, scalebs, rowsbs, sliceb,
        gsems, ssems, wsem, esem):
    cid = lax.axis_index("c")
    sid = lax.axis_index("s")

    @pl.loop(0, KC // L)
    def _(i):
      onesb[pl.ds(i * L, L)] = jnp.ones((L,), jnp.float32)

    f1 = pltpu.async_copy(z1_hbm, cnt_sp.at[pl.ds(sid * CSLICE, CSLICE)], esem)
    f2 = pltpu.async_copy(
        z2_hbm, acc_sp.at[pl.ds(sid * ROWS_PER_TILE, ROWS_PER_TILE)], esem)
    f1.wait()
    f2.wait()

    plsc.subcore_barrier()

    base1 = sid * ECT

    @pl.loop(0, ECT // MB)
    def _(m):
      off = base1 + m * MB
      f1 = pltpu.async_copy(ei_hbm.at[pl.ds(E + off, MB)], dstb, esem)
      f2 = pltpu.async_copy(typ_hbm.at[pl.ds(off, MB)], typb, esem)
      f1.wait()
      f2.wait()

      @pl.loop(0, MC)
      def _(c):
        for j in range(KC // L):
          sl = pl.ds(c * KC + j * L, L)
          db[c, pl.ds(j * L, L)] = typb[sl] * N + dstb[sl]

      @pl.loop(0, MC // 5)
      def _(w):
        descs = [
            pltpu.async_copy(onesb, cnt_sp.at[db.at[w * 5 + b]], wsem,
                             add=True)
            for b in range(5)
        ]
        for d_ in descs:
          d_.wait()

    plsc.subcore_barrier()

    cbase = sid * CSLICE
    for p in range(2):
      pltpu.sync_copy(cnt_sp.at[pl.ds(cbase + p * CHALF, CHALF)], sliceb)

      @pl.loop(0, CHALF // L)
      def _(i):
        sl = pl.ds(i * L, L)
        sliceb[sl] = 1.0 / jnp.maximum(sliceb[sl], 1.0)

      pltpu.sync_copy(sliceb, cnt_sp.at[pl.ds(cbase + p * CHALF, CHALF)])
    plsc.subcore_barrier()

    base2 = (cid * NS + sid) * EPT

    def chunk_in(c, b):
      ebase = c * KC
      sd = pltpu.async_copy(
          cnt_sp.at[typb.at[pl.ds(ebase, KC)]], scalebs[b], ssems[b])
      rd = pltpu.async_copy(
          z_hbm.at[srcb.at[pl.ds(ebase, KC)]], rowsbs[b], gsems[b])
      return sd, rd

    def chunk_out(c, b, sd, rd):
      sd.wait()
      rd.wait()
      rowsb = rowsbs[b]
      scaleb = scalebs[b]

      @pl.loop(0, KC // L)
      def _(gg):
        sv = scaleb[pl.ds(gg * L, L)]
        for j in range(L):
          s = sv[j]
          row = gg * L + j
          for q in range(D // L):
            sl = pl.ds(q * L, L)
            rowsb[row, sl] = rowsb[row, sl] * s

      return pltpu.async_copy(rowsb, acc_sp.at[db.at[c]], wsem, add=True)

    @pl.loop(0, EPT // MB)
    def _(m):
      off = base2 + m * MB
      f1 = pltpu.async_copy(ei_hbm.at[pl.ds(off, MB)], srcb, esem)
      f2 = pltpu.async_copy(ei_hbm.at[pl.ds(E + off, MB)], dstb, esem)
      f3 = pltpu.async_copy(typ_hbm.at[pl.ds(off, MB)], typb, esem)
      f1.wait()
      f2.wait()
      f3.wait()

      @pl.loop(0, MC)
      def _(c):
        for j in range(KC // L):
          sl = pl.ds(c * KC + j * L, L)
          t = typb[sl]
          d_ = dstb[sl]
          srcb[sl] = (t + 1) * N + srcb[sl]
          typb[sl] = t * N + d_
          db[c, pl.ds(j * L, L)] = d_

      @pl.loop(0, (MC - 1) // NBUF)
      def _(w):
        c0 = w * NBUF
        ins = [chunk_in(c0 + b, b) for b in range(NBUF)]
        outs = [chunk_out(c0 + b, b, *ins[b]) for b in range(NBUF)]
        for d_ in outs:
          d_.wait()

      sd, rd = chunk_in(MC - 1, 0)
      chunk_out(MC - 1, 0, sd, rd).wait()

    plsc.subcore_barrier()

    rbase = sid * ROWS_PER_TILE
    pltpu.sync_copy(acc_sp.at[pl.ds(rbase, ROWS_PER_TILE)],
                    parts_hbm.at[pl.ds(cid * NP + rbase, ROWS_PER_TILE)])

  z1 = jnp.zeros((CSLICE,), jnp.float32)
  z2 = jnp.zeros((ROWS_PER_TILE, D), jnp.float32)
  return k(edge_index.reshape(2 * E), etype, zflat, z1, z2)


BN = 2000


def _mm_body(x_ref, w_ref, b_ref, z_ref):
  i = pl.program_id(0)
  acc = jnp.dot(x_ref[...], w_ref[0], preferred_element_type=jnp.float32)
  sel = jnp.where(i == 0, 1.0, 0.0).astype(jnp.float32)
  z_ref[0] = acc + sel * b_ref[...]


def _fin_body(z0_ref, p_ref, a_ref, o_ref):
  o = z0_ref[0] + p_ref[0] + p_ref[1]
  o_ref[...] = jnp.where(o > 0, o, a_ref[...] * o)


def kernel(x, edge_index, edge_type, weight, root, bias, prelu_a):
  wcat = jnp.concatenate([root[None], weight], axis=0)

  zfull = pl.pallas_call(
      _mm_body,
      grid=(R + 1, N // BN),
      in_specs=[
          pl.BlockSpec((BN, D), lambda i, nb: (nb, 0)),
          pl.BlockSpec((1, D, D), lambda i, nb: (i, 0, 0)),
          pl.BlockSpec((1, D), lambda i, nb: (0, 0)),
      ],
      out_specs=pl.BlockSpec((1, BN, D), lambda i, nb: (i, nb, 0)),
      out_shape=jax.ShapeDtypeStruct((R + 1, N, D), jnp.float32),
  )(x, wcat, bias[None])

  zflat = zfull.reshape((R + 1) * N, D)
  parts = _sc_graph_kernel(edge_index, edge_type, zflat).reshape(NC, NP, D)

  return pl.pallas_call(
      _fin_body,
      grid=(N // BN,),
      in_specs=[
          pl.BlockSpec((1, BN, D), lambda nb: (0, nb, 0)),
          pl.BlockSpec((NC, BN, D), lambda nb: (0, nb, 0)),
          pl.BlockSpec((1, D), lambda nb: (0, 0)),
      ],
      out_specs=pl.BlockSpec((BN, D), lambda nb: (nb, 0)),
      out_shape=jax.ShapeDtypeStruct((N, D), jnp.float32),
  )(zfull, parts, prelu_a[None])


# speedup vs baseline: 1.1168x; 1.0222x over previous
import functools

import jax
import jax.numpy as jnp
from jax import lax
from jax.experimental import pallas as pl
from jax.experimental.pallas import tpu as pltpu
from jax.experimental.pallas import tpu_sc as plsc

NC = 2
NS = 16
NW = NC * NS
L = 16

N = 10000
E = 320000
D = 128
R = 8

CNTP = 81920
CSLICE = CNTP // NS
KC = 80
NBUF = 3
EPT = E // NW
ECT = E // NS
MB = 2000
MC = MB // KC
CHALF = CSLICE // 4
NP = 10240
ROWS_PER_TILE = NP // NS
ZROWS = 16


def _sc_graph_kernel(edge_index, etype, zflat):
  mesh = plsc.VectorSubcoreMesh(core_axis_name="c", subcore_axis_name="s")

  @functools.partial(
      pl.kernel,
      out_type=jax.ShapeDtypeStruct((NC * NP, D), jnp.float32),
      mesh=mesh,
      scratch_types=[
          pltpu.VMEM_SHARED((CNTP,), jnp.float32),
          pltpu.VMEM_SHARED((NP, D), jnp.float32),
          pltpu.VMEM((MB,), jnp.int32),
          pltpu.VMEM((MB,), jnp.int32),
          pltpu.VMEM((MB,), jnp.int32),
          pltpu.VMEM((MC, KC), jnp.int32),
          pltpu.VMEM((KC,), jnp.float32),
          [pltpu.VMEM((KC,), jnp.float32)] * NBUF,
          [pltpu.VMEM((KC, D), jnp.float32)] * NBUF,
          pltpu.VMEM((CHALF,), jnp.float32),
          [pltpu.SemaphoreType.DMA] * NBUF,
          [pltpu.SemaphoreType.DMA] * NBUF,
          pltpu.SemaphoreType.DMA,
          pltpu.SemaphoreType.DMA,
      ],
  )
  def k(ei_hbm, typ_hbm, z_hbm, z1_hbm, z2_hbm, parts_hbm,
        cnt_sp, acc_sp, srcb, dstb, typb, db,
        onesb, scalebs, rowsbs, sliceb,
        gsems, ssems, wsem, esem):
    cid = lax.axis_index("c")
    sid = lax.axis_index("s")

    @pl.loop(0, KC // L)
    def _(i):
      onesb[pl.ds(i * L, L)] = jnp.ones((L,), jnp.float32)

    f1 = pltpu.async_copy(z1_hbm, cnt_sp.at[pl.ds(sid * CSLICE, CSLICE)], esem)
    f2 = pltpu.async_copy(
        z2_hbm, acc_sp.at[pl.ds(sid * ROWS_PER_TILE, ROWS_PER_TILE)], esem)
    f1.wait()
    f2.wait()

    plsc.subcore_barrier()

    base1 = sid * ECT

    @pl.loop(0, ECT // MB)
    def _(m):
      off = base1 + m * MB
      f1 = pltpu.async_copy(ei_hbm.at[pl.ds(E + off, MB)], dstb, esem)
      f2 = pltpu.async_copy(typ_hbm.at[pl.ds(off, MB)], typb, esem)
      f1.wait()
      f2.wait()

      @pl.loop(0, MC)
      def _(c):
        for j in range(KC // L):
          sl = pl.ds(c * KC + j * L, L)
          db[c, pl.ds(j * L, L)] = typb[sl] * N + dstb[sl]

      @pl.loop(0, MC // 5)
      def _(w):
        descs = [
            pltpu.async_copy(onesb, cnt_sp.at[db.at[w * 5 + b]], wsem,
                             add=True)
            for b in range(5)
        ]
        for d_ in descs:
          d_.wait()

    plsc.subcore_barrier()

    cbase = sid * CSLICE
    for p in range(4):
      pltpu.sync_copy(cnt_sp.at[pl.ds(cbase + p * CHALF, CHALF)], sliceb)

      @pl.loop(0, CHALF // L)
      def _(i):
        sl = pl.ds(i * L, L)
        sliceb[sl] = 1.0 / jnp.maximum(sliceb[sl], 1.0)

      pltpu.sync_copy(sliceb, cnt_sp.at[pl.ds(cbase + p * CHALF, CHALF)])
    plsc.subcore_barrier()

    base2 = (cid * NS + sid) * EPT

    def chunk_in(c, b):
      ebase = c * KC
      sd = pltpu.async_copy(
          cnt_sp.at[typb.at[pl.ds(ebase, KC)]], scalebs[b], ssems[b])
      rd = pltpu.async_copy(
          z_hbm.at[srcb.at[pl.ds(ebase, KC)]], rowsbs[b], gsems[b])
      return sd, rd

    def chunk_out(c, b, sd, rd):
      sd.wait()
      rd.wait()
      rowsb = rowsbs[b]
      scaleb = scalebs[b]

      @pl.loop(0, KC // L)
      def _(gg):
        sv = scaleb[pl.ds(gg * L, L)]
        for j in range(L):
          s = sv[j]
          row = gg * L + j
          for q in range(D // L):
            sl = pl.ds(q * L, L)
            rowsb[row, sl] = rowsb[row, sl] * s

      return pltpu.async_copy(rowsb, acc_sp.at[db.at[c]], wsem, add=True)

    @pl.loop(0, EPT // MB)
    def _(m):
      off = base2 + m * MB
      f1 = pltpu.async_copy(ei_hbm.at[pl.ds(off, MB)], srcb, esem)
      f2 = pltpu.async_copy(ei_hbm.at[pl.ds(E + off, MB)], dstb, esem)
      f3 = pltpu.async_copy(typ_hbm.at[pl.ds(off, MB)], typb, esem)
      f1.wait()
      f2.wait()
      f3.wait()

      @pl.loop(0, MC)
      def _(c):
        for j in range(KC // L):
          sl = pl.ds(c * KC + j * L, L)
          t = typb[sl]
          d_ = dstb[sl]
          srcb[sl] = (t + 1) * N + srcb[sl]
          typb[sl] = t * N + d_
          db[c, pl.ds(j * L, L)] = d_

      @pl.loop(0, (MC - 1) // (2 * NBUF))
      def _(w):
        c0 = w * 2 * NBUF
        ins = [chunk_in(c0 + b, b) for b in range(NBUF)]
        outs = [chunk_out(c0 + b, b, *ins[b]) for b in range(NBUF)]
        ins2 = []
        for b in range(NBUF):
          outs[b].wait()
          ins2.append(chunk_in(c0 + NBUF + b, b))
        outs2 = [chunk_out(c0 + NBUF + b, b, *ins2[b]) for b in range(NBUF)]
        for d_ in outs2:
          d_.wait()

      sd, rd = chunk_in(MC - 1, 0)
      chunk_out(MC - 1, 0, sd, rd).wait()

    plsc.subcore_barrier()

    rbase = sid * ROWS_PER_TILE
    pltpu.sync_copy(acc_sp.at[pl.ds(rbase, ROWS_PER_TILE)],
                    parts_hbm.at[pl.ds(cid * NP + rbase, ROWS_PER_TILE)])

  z1 = jnp.zeros((CSLICE,), jnp.float32)
  z2 = jnp.zeros((ROWS_PER_TILE, D), jnp.float32)
  return k(edge_index.reshape(2 * E), etype, zflat, z1, z2)


BN = 2000


def _mm_body(x_ref, w_ref, b_ref, z_ref):
  i = pl.program_id(0)
  acc = jnp.dot(x_ref[...], w_ref[0], preferred_element_type=jnp.float32)
  sel = jnp.where(i == 0, 1.0, 0.0).astype(jnp.float32)
  z_ref[0] = acc + sel * b_ref[...]


def _fin_body(z0_ref, p_ref, a_ref, o_ref):
  o = z0_ref[0] + p_ref[0] + p_ref[1]
  o_ref[...] = jnp.where(o > 0, o, a_ref[...] * o)


def kernel(x, edge_index, edge_type, weight, root, bias, prelu_a):
  wcat = jnp.concatenate([root[None], weight], axis=0)

  zfull = pl.pallas_call(
      _mm_body,
      grid=(R + 1, N // BN),
      in_specs=[
          pl.BlockSpec((BN, D), lambda i, nb: (nb, 0)),
          pl.BlockSpec((1, D, D), lambda i, nb: (i, 0, 0)),
          pl.BlockSpec((1, D), lambda i, nb: (0, 0)),
      ],
      out_specs=pl.BlockSpec((1, BN, D), lambda i, nb: (i, nb, 0)),
      out_shape=jax.ShapeDtypeStruct((R + 1, N, D), jnp.float32),
  )(x, wcat, bias[None])

  zflat = zfull.reshape((R + 1) * N, D)
  parts = _sc_graph_kernel(edge_index, edge_type, zflat).reshape(NC, NP, D)

  return pl.pallas_call(
      _fin_body,
      grid=(N // BN,),
      in_specs=[
          pl.BlockSpec((1, BN, D), lambda nb: (0, nb, 0)),
          pl.BlockSpec((NC, BN, D), lambda nb: (0, nb, 0)),
          pl.BlockSpec((1, D), lambda nb: (0, 0)),
      ],
      out_specs=pl.BlockSpec((BN, D), lambda nb: (nb, 0)),
      out_shape=jax.ShapeDtypeStruct((N, D), jnp.float32),
  )(zfull, parts, prelu_a[None])


# speedup vs baseline: 1.2306x; 1.1018x over previous
import functools

import jax
import jax.numpy as jnp
from jax import lax
from jax.experimental import pallas as pl
from jax.experimental.pallas import tpu as pltpu
from jax.experimental.pallas import tpu_sc as plsc

NC = 2
NS = 16
NW = NC * NS
L = 16

N = 10000
E = 320000
D = 128
R = 8

CNTP = 81920
CSLICE = CNTP // NS
KC = 80
NBUF = 3
EPT = E // NW
ECT = E // NS
MB = 2000
MC = MB // KC
CHALF = CSLICE // 4
NP = 10240
ROWS_PER_TILE = NP // NS
ZROWS = 16


def _sc_count_kernel(edge_index, etype):
  mesh = plsc.VectorSubcoreMesh(core_axis_name="c", subcore_axis_name="s")

  @functools.partial(
      pl.kernel,
      out_type=jax.ShapeDtypeStruct((NC, CNTP), jnp.float32),
      mesh=mesh,
      scratch_types=[
          pltpu.VMEM_SHARED((CNTP,), jnp.float32),
          pltpu.VMEM((MB,), jnp.int32),
          pltpu.VMEM((MB,), jnp.int32),
          pltpu.VMEM((MC, KC), jnp.int32),
          pltpu.VMEM((KC,), jnp.float32),
          pltpu.VMEM((CHALF,), jnp.float32),
          pltpu.SemaphoreType.DMA,
          pltpu.SemaphoreType.DMA,
      ],
  )
  def k(ei_hbm, typ_hbm, out_hbm, cnt_sp, dstb, typb, db, onesb, sliceb,
        wsem, esem):
    cid = lax.axis_index("c")
    sid = lax.axis_index("s")
    zeros = jnp.zeros((L,), jnp.float32)

    @pl.loop(0, CHALF // L)
    def _(i):
      sliceb[pl.ds(i * L, L)] = zeros

    @pl.loop(0, KC // L)
    def _(i):
      onesb[pl.ds(i * L, L)] = jnp.ones((L,), jnp.float32)

    for p in range(4):
      pltpu.sync_copy(sliceb,
                      cnt_sp.at[pl.ds(sid * CSLICE + p * CHALF, CHALF)])
    plsc.subcore_barrier()

    base1 = (cid * NS + sid) * EPT

    @pl.loop(0, EPT // MB)
    def _(m):
      off = base1 + m * MB
      f1 = pltpu.async_copy(ei_hbm.at[pl.ds(E + off, MB)], dstb, esem)
      f2 = pltpu.async_copy(typ_hbm.at[pl.ds(off, MB)], typb, esem)
      f1.wait()
      f2.wait()

      @pl.loop(0, MC)
      def _(c):
        for j in range(KC // L):
          sl = pl.ds(c * KC + j * L, L)
          db[c, pl.ds(j * L, L)] = typb[sl] * N + dstb[sl]

      @pl.loop(0, MC // 5)
      def _(w):
        descs = [
            pltpu.async_copy(onesb, cnt_sp.at[db.at[w * 5 + b]], wsem,
                             add=True)
            for b in range(5)
        ]
        for d_ in descs:
          d_.wait()

    plsc.subcore_barrier()
    pltpu.sync_copy(cnt_sp.at[pl.ds(sid * CSLICE, CSLICE)],
                    out_hbm.at[cid, pl.ds(sid * CSLICE, CSLICE)])

  return k(edge_index, etype)


def _sc_graph_kernel(edge_index, etype, zflat, cnts):
  mesh = plsc.VectorSubcoreMesh(core_axis_name="c", subcore_axis_name="s")

  @functools.partial(
      pl.kernel,
      out_type=jax.ShapeDtypeStruct((NC * NP, D), jnp.float32),
      mesh=mesh,
      scratch_types=[
          pltpu.VMEM_SHARED((CNTP,), jnp.float32),
          pltpu.VMEM_SHARED((NP, D), jnp.float32),
          pltpu.VMEM((MB,), jnp.int32),
          pltpu.VMEM((MB,), jnp.int32),
          pltpu.VMEM((MB,), jnp.int32),
          pltpu.VMEM((MC, KC), jnp.int32),
          [pltpu.VMEM((KC,), jnp.float32)] * NBUF,
          [pltpu.VMEM((KC, D), jnp.float32)] * NBUF,
          pltpu.VMEM((CHALF,), jnp.float32),
          pltpu.VMEM((CHALF,), jnp.float32),
          [pltpu.SemaphoreType.DMA] * NBUF,
          [pltpu.SemaphoreType.DMA] * NBUF,
          pltpu.SemaphoreType.DMA,
          pltpu.SemaphoreType.DMA,
      ],
  )
  def k(ei_hbm, typ_hbm, z_hbm, cnts_hbm, z2_hbm, parts_hbm,
        cnt_sp, acc_sp, srcb, dstb, typb, db,
        scalebs, rowsbs, sliceb, slice2b,
        gsems, ssems, wsem, esem):
    cid = lax.axis_index("c")
    sid = lax.axis_index("s")

    f2 = pltpu.async_copy(
        z2_hbm, acc_sp.at[pl.ds(sid * ROWS_PER_TILE, ROWS_PER_TILE)], esem)

    cbase = sid * CSLICE
    for p in range(4):
      o = cbase + p * CHALF
      g1 = pltpu.async_copy(cnts_hbm.at[0, pl.ds(o, CHALF)], sliceb, esem)
      g2 = pltpu.async_copy(cnts_hbm.at[1, pl.ds(o, CHALF)], slice2b, esem)
      g1.wait()
      g2.wait()

      @pl.loop(0, CHALF // L)
      def _(i):
        sl = pl.ds(i * L, L)
        c_ = sliceb[sl] + slice2b[sl]
        sliceb[sl] = 1.0 / jnp.maximum(c_, 1.0)

      pltpu.sync_copy(sliceb, cnt_sp.at[pl.ds(o, CHALF)])
    f2.wait()
    plsc.subcore_barrier()

    base2 = (cid * NS + sid) * EPT

    def chunk_in(c, b):
      ebase = c * KC
      sd = pltpu.async_copy(
          cnt_sp.at[typb.at[pl.ds(ebase, KC)]], scalebs[b], ssems[b])
      rd = pltpu.async_copy(
          z_hbm.at[srcb.at[pl.ds(ebase, KC)]], rowsbs[b], gsems[b])
      return sd, rd

    def chunk_out(c, b, sd, rd):
      sd.wait()
      rd.wait()
      rowsb = rowsbs[b]
      scaleb = scalebs[b]

      @pl.loop(0, KC // L)
      def _(gg):
        sv = scaleb[pl.ds(gg * L, L)]
        for j in range(L):
          s = sv[j]
          row = gg * L + j
          for q in range(D // L):
            sl = pl.ds(q * L, L)
            rowsb[row, sl] = rowsb[row, sl] * s

      return pltpu.async_copy(rowsb, acc_sp.at[db.at[c]], wsem, add=True)

    @pl.loop(0, EPT // MB)
    def _(m):
      off = base2 + m * MB
      f1 = pltpu.async_copy(ei_hbm.at[pl.ds(off, MB)], srcb, esem)
      f2 = pltpu.async_copy(ei_hbm.at[pl.ds(E + off, MB)], dstb, esem)
      f3 = pltpu.async_copy(typ_hbm.at[pl.ds(off, MB)], typb, esem)
      f1.wait()
      f2.wait()
      f3.wait()

      @pl.loop(0, MC)
      def _(c):
        for j in range(KC // L):
          sl = pl.ds(c * KC + j * L, L)
          t = typb[sl]
          d_ = dstb[sl]
          srcb[sl] = (t + 1) * N + srcb[sl]
          typb[sl] = t * N + d_
          db[c, pl.ds(j * L, L)] = d_

      @pl.loop(0, (MC - 1) // (2 * NBUF))
      def _(w):
        c0 = w * 2 * NBUF
        ins = [chunk_in(c0 + b, b) for b in range(NBUF)]
        outs = [chunk_out(c0 + b, b, *ins[b]) for b in range(NBUF)]
        ins2 = []
        for b in range(NBUF):
          outs[b].wait()
          ins2.append(chunk_in(c0 + NBUF + b, b))
        outs2 = [chunk_out(c0 + NBUF + b, b, *ins2[b]) for b in range(NBUF)]
        for d_ in outs2:
          d_.wait()

      sd, rd = chunk_in(MC - 1, 0)
      chunk_out(MC - 1, 0, sd, rd).wait()

    plsc.subcore_barrier()

    rbase = sid * ROWS_PER_TILE
    pltpu.sync_copy(acc_sp.at[pl.ds(rbase, ROWS_PER_TILE)],
                    parts_hbm.at[pl.ds(cid * NP + rbase, ROWS_PER_TILE)])

  z2 = jnp.zeros((ROWS_PER_TILE, D), jnp.float32)
  return k(edge_index, etype, zflat, cnts, z2)


BN = 2000


def _mm_body(x_ref, w_ref, b_ref, z_ref):
  i = pl.program_id(0)
  acc = jnp.dot(x_ref[...], w_ref[0], preferred_element_type=jnp.float32)
  sel = jnp.where(i == 0, 1.0, 0.0).astype(jnp.float32)
  z_ref[0] = acc + sel * b_ref[...]


def _fin_body(z0_ref, p_ref, a_ref, o_ref):
  o = z0_ref[0] + p_ref[0] + p_ref[1]
  o_ref[...] = jnp.where(o > 0, o, a_ref[...] * o)


def kernel(x, edge_index, edge_type, weight, root, bias, prelu_a):
  wcat = jnp.concatenate([root[None], weight], axis=0)

  zfull = pl.pallas_call(
      _mm_body,
      grid=(R + 1, N // BN),
      in_specs=[
          pl.BlockSpec((BN, D), lambda i, nb: (nb, 0)),
          pl.BlockSpec((1, D, D), lambda i, nb: (i, 0, 0)),
          pl.BlockSpec((1, D), lambda i, nb: (0, 0)),
      ],
      out_specs=pl.BlockSpec((1, BN, D), lambda i, nb: (i, nb, 0)),
      out_shape=jax.ShapeDtypeStruct((R + 1, N, D), jnp.float32),
  )(x, wcat, bias[None])

  ei_flat = edge_index.reshape(2 * E)
  cnts = _sc_count_kernel(ei_flat, edge_type)
  zflat = zfull.reshape((R + 1) * N, D)
  parts = _sc_graph_kernel(ei_flat, edge_type, zflat, cnts).reshape(NC, NP, D)

  return pl.pallas_call(
      _fin_body,
      grid=(N // BN,),
      in_specs=[
          pl.BlockSpec((1, BN, D), lambda nb: (0, nb, 0)),
          pl.BlockSpec((NC, BN, D), lambda nb: (0, nb, 0)),
          pl.BlockSpec((1, D), lambda nb: (0, 0)),
      ],
      out_specs=pl.BlockSpec((BN, D), lambda nb: (nb, 0)),
      out_shape=jax.ShapeDtypeStruct((N, D), jnp.float32),
  )(zfull, parts, prelu_a[None])


# speedup vs baseline: 1.2310x; 1.0004x over previous
import functools

import jax
import jax.numpy as jnp
from jax import lax
from jax.experimental import pallas as pl
from jax.experimental.pallas import tpu as pltpu
from jax.experimental.pallas import tpu_sc as plsc

NC = 2
NS = 16
NW = NC * NS
L = 16

N = 10000
E = 320000
D = 128
R = 8

CNTP = 81920
CSLICE = CNTP // NS
KC = 80
NBUF = 3
EPT = E // NW
ECT = E // NS
MB = 2000
MC = MB // KC
CHALF = CSLICE // 4
NP = 10240
ROWS_PER_TILE = NP // NS
ZROWS = 16


def _sc_count_kernel(edge_index, etype):
  mesh = plsc.VectorSubcoreMesh(core_axis_name="c", subcore_axis_name="s")

  @functools.partial(
      pl.kernel,
      out_type=jax.ShapeDtypeStruct((NC * CNTP,), jnp.float32),
      mesh=mesh,
      scratch_types=[
          pltpu.VMEM_SHARED((CNTP,), jnp.float32),
          pltpu.VMEM((MB,), jnp.int32),
          pltpu.VMEM((MB,), jnp.int32),
          pltpu.VMEM((MC, KC), jnp.int32),
          pltpu.VMEM((KC,), jnp.float32),
          pltpu.VMEM((CHALF,), jnp.float32),
          pltpu.SemaphoreType.DMA,
          pltpu.SemaphoreType.DMA,
      ],
  )
  def k(ei_hbm, typ_hbm, out_hbm, cnt_sp, dstb, typb, db, onesb, sliceb,
        wsem, esem):
    cid = lax.axis_index("c")
    sid = lax.axis_index("s")
    zeros = jnp.zeros((L,), jnp.float32)

    @pl.loop(0, CHALF // L)
    def _(i):
      sliceb[pl.ds(i * L, L)] = zeros

    @pl.loop(0, KC // L)
    def _(i):
      onesb[pl.ds(i * L, L)] = jnp.ones((L,), jnp.float32)

    for p in range(4):
      pltpu.sync_copy(sliceb,
                      cnt_sp.at[pl.ds(sid * CSLICE + p * CHALF, CHALF)])
    plsc.subcore_barrier()

    base1 = (cid * NS + sid) * EPT

    @pl.loop(0, EPT // MB)
    def _(m):
      off = base1 + m * MB
      f1 = pltpu.async_copy(ei_hbm.at[pl.ds(E + off, MB)], dstb, esem)
      f2 = pltpu.async_copy(typ_hbm.at[pl.ds(off, MB)], typb, esem)
      f1.wait()
      f2.wait()

      @pl.loop(0, MC)
      def _(c):
        for j in range(KC // L):
          sl = pl.ds(c * KC + j * L, L)
          db[c, pl.ds(j * L, L)] = typb[sl] * N + dstb[sl]

      @pl.loop(0, MC // 5)
      def _(w):
        descs = [
            pltpu.async_copy(onesb, cnt_sp.at[db.at[w * 5 + b]], wsem,
                             add=True)
            for b in range(5)
        ]
        for d_ in descs:
          d_.wait()

    plsc.subcore_barrier()
    pltpu.sync_copy(cnt_sp.at[pl.ds(sid * CSLICE, CSLICE)],
                    out_hbm.at[pl.ds(cid * CNTP + sid * CSLICE, CSLICE)])

  return k(edge_index, etype)


def _sc_graph_kernel(edge_index, etype, zflat, cnts):
  mesh = plsc.VectorSubcoreMesh(core_axis_name="c", subcore_axis_name="s")

  @functools.partial(
      pl.kernel,
      out_type=jax.ShapeDtypeStruct((NC * NP, D), jnp.float32),
      mesh=mesh,
      scratch_types=[
          pltpu.VMEM_SHARED((CNTP,), jnp.float32),
          pltpu.VMEM_SHARED((NP, D), jnp.float32),
          pltpu.VMEM((MB,), jnp.int32),
          pltpu.VMEM((MB,), jnp.int32),
          pltpu.VMEM((MB,), jnp.int32),
          pltpu.VMEM((MC, KC), jnp.int32),
          [pltpu.VMEM((KC,), jnp.float32)] * NBUF,
          [pltpu.VMEM((KC, D), jnp.float32)] * NBUF,
          pltpu.VMEM((CHALF,), jnp.float32),
          pltpu.VMEM((CHALF,), jnp.float32),
          [pltpu.SemaphoreType.DMA] * NBUF,
          [pltpu.SemaphoreType.DMA] * NBUF,
          pltpu.SemaphoreType.DMA,
          pltpu.SemaphoreType.DMA,
      ],
  )
  def k(ei_hbm, typ_hbm, z_hbm, cnts_hbm, z2_hbm, parts_hbm,
        cnt_sp, acc_sp, srcb, dstb, typb, db,
        scalebs, rowsbs, sliceb, slice2b,
        gsems, ssems, wsem, esem):
    cid = lax.axis_index("c")
    sid = lax.axis_index("s")

    f2 = pltpu.async_copy(
        z2_hbm, acc_sp.at[pl.ds(sid * ROWS_PER_TILE, ROWS_PER_TILE)], wsem)

    cbase = sid * CSLICE
    for p in range(4):
      o = cbase + p * CHALF
      g1 = pltpu.async_copy(cnts_hbm.at[pl.ds(o, CHALF)], sliceb, esem)
      g2 = pltpu.async_copy(cnts_hbm.at[pl.ds(CNTP + o, CHALF)], slice2b,
                            gsems[0])
      g1.wait()
      g2.wait()

      @pl.loop(0, CHALF // L)
      def _(i):
        sl = pl.ds(i * L, L)
        c_ = sliceb[sl] + slice2b[sl]
        sliceb[sl] = 1.0 / jnp.maximum(c_, 1.0)

      pltpu.sync_copy(sliceb, cnt_sp.at[pl.ds(o, CHALF)])
    f2.wait()
    plsc.subcore_barrier()

    base2 = (cid * NS + sid) * EPT

    def chunk_in(c, b):
      ebase = c * KC
      sd = pltpu.async_copy(
          cnt_sp.at[typb.at[pl.ds(ebase, KC)]], scalebs[b], ssems[b])
      rd = pltpu.async_copy(
          z_hbm.at[srcb.at[pl.ds(ebase, KC)]], rowsbs[b], gsems[b])
      return sd, rd

    def chunk_out(c, b, sd, rd):
      sd.wait()
      rd.wait()
      rowsb = rowsbs[b]
      scaleb = scalebs[b]

      @pl.loop(0, KC // L)
      def _(gg):
        sv = scaleb[pl.ds(gg * L, L)]
        for j in range(L):
          s = sv[j]
          row = gg * L + j
          for q in range(D // L):
            sl = pl.ds(q * L, L)
            rowsb[row, sl] = rowsb[row, sl] * s

      return pltpu.async_copy(rowsb, acc_sp.at[db.at[c]], wsem, add=True)

    @pl.loop(0, EPT // MB)
    def _(m):
      off = base2 + m * MB
      f1 = pltpu.async_copy(ei_hbm.at[pl.ds(off, MB)], srcb, esem)
      f2 = pltpu.async_copy(ei_hbm.at[pl.ds(E + off, MB)], dstb, esem)
      f3 = pltpu.async_copy(typ_hbm.at[pl.ds(off, MB)], typb, esem)
      f1.wait()
      f2.wait()
      f3.wait()

      @pl.loop(0, MC)
      def _(c):
        for j in range(KC // L):
          sl = pl.ds(c * KC + j * L, L)
          t = typb[sl]
          d_ = dstb[sl]
          srcb[sl] = (t + 1) * N + srcb[sl]
          typb[sl] = t * N + d_
          db[c, pl.ds(j * L, L)] = d_

      @pl.loop(0, (MC - 1) // (2 * NBUF))
      def _(w):
        c0 = w * 2 * NBUF
        ins = [chunk_in(c0 + b, b) for b in range(NBUF)]
        outs = [chunk_out(c0 + b, b, *ins[b]) for b in range(NBUF)]
        ins2 = []
        for b in range(NBUF):
          outs[b].wait()
          ins2.append(chunk_in(c0 + NBUF + b, b))
        outs2 = [chunk_out(c0 + NBUF + b, b, *ins2[b]) for b in range(NBUF)]
        for d_ in outs2:
          d_.wait()

      sd, rd = chunk_in(MC - 1, 0)
      chunk_out(MC - 1, 0, sd, rd).wait()

    plsc.subcore_barrier()

    rbase = sid * ROWS_PER_TILE
    pltpu.sync_copy(acc_sp.at[pl.ds(rbase, ROWS_PER_TILE)],
                    parts_hbm.at[pl.ds(cid * NP + rbase, ROWS_PER_TILE)])

  z2 = jnp.zeros((ROWS_PER_TILE, D), jnp.float32)
  return k(edge_index, etype, zflat, cnts, z2)


BN = 2000


def _mm_body(x_ref, w_ref, b_ref, z_ref):
  i = pl.program_id(0)
  acc = jnp.dot(x_ref[...], w_ref[0], preferred_element_type=jnp.float32)
  sel = jnp.where(i == 0, 1.0, 0.0).astype(jnp.float32)
  z_ref[0] = acc + sel * b_ref[...]


def _fin_body(z0_ref, p_ref, a_ref, o_ref):
  o = z0_ref[0] + p_ref[0] + p_ref[1]
  o_ref[...] = jnp.where(o > 0, o, a_ref[...] * o)


def kernel(x, edge_index, edge_type, weight, root, bias, prelu_a):
  wcat = jnp.concatenate([root[None], weight], axis=0)

  zfull = pl.pallas_call(
      _mm_body,
      grid=(R + 1, N // BN),
      in_specs=[
          pl.BlockSpec((BN, D), lambda i, nb: (nb, 0)),
          pl.BlockSpec((1, D, D), lambda i, nb: (i, 0, 0)),
          pl.BlockSpec((1, D), lambda i, nb: (0, 0)),
      ],
      out_specs=pl.BlockSpec((1, BN, D), lambda i, nb: (i, nb, 0)),
      out_shape=jax.ShapeDtypeStruct((R + 1, N, D), jnp.float32),
  )(x, wcat, bias[None])

  ei_flat = edge_index.reshape(2 * E)
  cnts = _sc_count_kernel(ei_flat, edge_type)
  zflat = zfull.reshape((R + 1) * N, D)
  parts = _sc_graph_kernel(ei_flat, edge_type, zflat, cnts).reshape(NC, NP, D)

  return pl.pallas_call(
      _fin_body,
      grid=(N // BN,),
      in_specs=[
          pl.BlockSpec((1, BN, D), lambda nb: (0, nb, 0)),
          pl.BlockSpec((NC, BN, D), lambda nb: (0, nb, 0)),
          pl.BlockSpec((1, D), lambda nb: (0, 0)),
      ],
      out_specs=pl.BlockSpec((BN, D), lambda nb: (nb, 0)),
      out_shape=jax.ShapeDtypeStruct((N, D), jnp.float32),
  )(zfull, parts, prelu_a[None])
